# Initial kernel scaffold; baseline (speedup 1.0000x reference)
import jax
import jax.numpy as jnp
from jax.experimental import pallas as pl


def kernel(news_text, user_history_batch, user_ids, candidate_news_ids, emb_table, user_table, Wself, bself, Waggr, baggr):
    raise NotImplementedError("write your pallas kernel here")


# speedup vs baseline: 6.3092x; 6.3092x over previous
import functools

import jax
import jax.numpy as jnp
from jax import lax
from jax.experimental import pallas as pl
from jax.experimental.pallas import tpu as pltpu
from jax.experimental.pallas import tpu_sc as plsc

NUM_NEWS = 100000
MAX_LEN = 20
D = 64
B = 4096
HIST = 50

NC = 2
NS = 16
NW = NC * NS
L = 16


C1 = 16
NCHUNK1 = NUM_NEWS // C1
JMAX1 = (NCHUNK1 + NW - 1) // NW
IDX_SLICE = 80
NSLICE1 = (C1 * MAX_LEN) // IDX_SLICE


def _news_pool_body(text_hbm, table_hbm, out_hbm, idx_v, rows_v,
                    out_v, sem):
  wid = lax.axis_index("s") * NC + lax.axis_index("c")

  lanes = lax.iota(jnp.int32, L)

  def chunk_body(j, _):
    g = j * NW + wid

    @pl.when(g < NCHUNK1)
    def _():
      pltpu.sync_copy(text_hbm.at[pl.ds(g * C1 * MAX_LEN, C1 * MAX_LEN)],
                      idx_v)
      descs = []
      for k in range(NSLICE1):
        descs.append(pltpu.async_copy(
            table_hbm.at[idx_v.at[pl.ds(k * IDX_SLICE, IDX_SLICE)]],
            rows_v.at[pl.ds(k * IDX_SLICE, IDX_SLICE)], sem))
      for d in descs:
        d.wait()

      def row_body(c, _):
        base = c * MAX_LEN
        a = idx_v[pl.ds(base, L)]
        b = idx_v[pl.ds(base + 4, L)]
        ones = jnp.ones((L,), jnp.float32)
        zeros = jnp.zeros((L,), jnp.float32)
        cnt = (jnp.sum(jnp.where(a != 0, ones, zeros)) +
               jnp.sum(jnp.where((b != 0) & (lanes >= 12), ones, zeros)))
        s = ones / jnp.maximum(jnp.full((L,), cnt), 1e-9)
        for d in range(D // L):
          acc = rows_v[base, pl.ds(d * L, L)]
          for t in range(1, MAX_LEN):
            acc = acc + rows_v[base + t, pl.ds(d * L, L)]
          out_v[c, pl.ds(d * L, L)] = acc * s
        return 0

      lax.fori_loop(0, C1, row_body, 0)
      pltpu.sync_copy(out_v, out_hbm.at[pl.ds(g * C1, C1)])

    return 0

  lax.fori_loop(0, JMAX1, chunk_body, 0)


@functools.partial(jax.jit, static_argnames=())
def _news_pool(news_text_flat, emb_table):
  mesh = plsc.VectorSubcoreMesh(core_axis_name="c", subcore_axis_name="s")
  kern = pl.kernel(
      _news_pool_body,
      out_type=jax.ShapeDtypeStruct((NUM_NEWS, D), jnp.float32),
      mesh=mesh,
      compiler_params=pltpu.CompilerParams(needs_layout_passes=False, use_tc_tiling_on_sc=False),
      scratch_types=[
          pltpu.VMEM((C1 * MAX_LEN,), jnp.int32),
          pltpu.VMEM((C1 * MAX_LEN, D), jnp.float32),
          pltpu.VMEM((C1, D), jnp.float32),
          pltpu.SemaphoreType.DMA,
      ],
  )
  return kern(news_text_flat, emb_table)


C2 = 16
NCHUNK2 = B // C2
JMAX2 = NCHUNK2 // NW
NSLICE2 = (C2 * HIST) // IDX_SLICE


def _user_gather_body(hist_hbm, uid_hbm, cand_hbm, nemb_hbm, utab_hbm,
                      hrep_hbm, uself_hbm, cemb_hbm,
                      idxh_v, idxu_v, idxc_v, rows_v, urow_v, crow_v,
                      out_v, sem):
  wid = lax.axis_index("s") * NC + lax.axis_index("c")

  def chunk_body(j, _):
    g = j * NW + wid
    pltpu.sync_copy(hist_hbm.at[pl.ds(g * C2 * HIST, C2 * HIST)], idxh_v)
    pltpu.sync_copy(uid_hbm.at[pl.ds(g * C2, C2)], idxu_v)
    pltpu.sync_copy(cand_hbm.at[pl.ds(g * C2, C2)], idxc_v)
    descs = []
    for k in range(NSLICE2):
      descs.append(pltpu.async_copy(
          nemb_hbm.at[idxh_v.at[pl.ds(k * IDX_SLICE, IDX_SLICE)]],
          rows_v.at[pl.ds(k * IDX_SLICE, IDX_SLICE)], sem))
    descs.append(pltpu.async_copy(utab_hbm.at[idxu_v], urow_v, sem))
    descs.append(pltpu.async_copy(nemb_hbm.at[idxc_v], crow_v, sem))
    for d in descs:
      d.wait()

    def row_body(c, _):
      base = c * HIST
      for d in range(D // L):
        acc = rows_v[base, pl.ds(d * L, L)]
        for t in range(1, HIST):
          acc = acc + rows_v[base + t, pl.ds(d * L, L)]
        out_v[c, pl.ds(d * L, L)] = acc * (1.0 / HIST)
      return 0

    lax.fori_loop(0, C2, row_body, 0)
    pltpu.sync_copy(out_v, hrep_hbm.at[pl.ds(g * C2, C2)])
    pltpu.sync_copy(urow_v, uself_hbm.at[pl.ds(g * C2, C2)])
    pltpu.sync_copy(crow_v, cemb_hbm.at[pl.ds(g * C2, C2)])
    return 0

  lax.fori_loop(0, JMAX2, chunk_body, 0)


@jax.jit
def _user_gather(hist_flat, user_ids, cand_ids, news_emb, user_table):
  mesh = plsc.VectorSubcoreMesh(core_axis_name="c", subcore_axis_name="s")
  sds = jax.ShapeDtypeStruct((B, D), jnp.float32)
  kern = pl.kernel(
      _user_gather_body,
      out_type=(sds, sds, sds),
      mesh=mesh,
      compiler_params=pltpu.CompilerParams(needs_layout_passes=False, use_tc_tiling_on_sc=False),
      scratch_types=[
          pltpu.VMEM((C2 * HIST,), jnp.int32),
          pltpu.VMEM((C2,), jnp.int32),
          pltpu.VMEM((C2,), jnp.int32),
          pltpu.VMEM((C2 * HIST, D), jnp.float32),
          pltpu.VMEM((C2, D), jnp.float32),
          pltpu.VMEM((C2, D), jnp.float32),
          pltpu.VMEM((C2, D), jnp.float32),
          pltpu.SemaphoreType.DMA,
      ],
  )
  return kern(hist_flat, user_ids, cand_ids, news_emb, user_table)


def _dense_body(u_ref, h_ref, c_ref, ws_ref, bs_ref, wa_ref, ba_ref, o_ref):
  dn = (((1,), (1,)), ((), ()))
  x = lax.dot_general(u_ref[...], ws_ref[...], dn,
                      preferred_element_type=jnp.float32)
  y = lax.dot_general(h_ref[...], wa_ref[...], dn,
                      preferred_element_type=jnp.float32)
  z = jnp.tanh(x + y + (bs_ref[...] + ba_ref[...])[None, :])
  o_ref[...] = jnp.sum(z * c_ref[...], axis=1)


@jax.jit
def _dense_score(u_self, hist_rep, cand_emb, Wself, bself, Waggr, baggr):
  return pl.pallas_call(
      _dense_body,
      out_shape=jax.ShapeDtypeStruct((B,), jnp.float32),
  )(u_self, hist_rep, cand_emb, Wself, bself, Waggr, baggr)


def kernel(news_text, user_history_batch, user_ids, candidate_news_ids,
           emb_table, user_table, Wself, bself, Waggr, baggr):
  news_text_flat = jnp.reshape(news_text.astype(jnp.int32), (-1,))
  hist_flat = jnp.reshape(user_history_batch.astype(jnp.int32), (-1,))
  user_ids = user_ids.astype(jnp.int32)
  cand_ids = candidate_news_ids.astype(jnp.int32)

  news_emb = _news_pool(news_text_flat, emb_table)
  hist_rep, u_self, cand_emb = _user_gather(
      hist_flat, user_ids, cand_ids, news_emb, user_table)
  return _dense_score(u_self, hist_rep, cand_emb, Wself, bself, Waggr, baggr)
